# baseline (speedup 1.0000x reference)
import jax
import jax.numpy as jnp
from jax import lax
from jax.experimental import pallas as pl
from jax.experimental.pallas import tpu as pltpu
from jax.experimental.pallas import tpu_sc as plsc

_NC = 2
_NS = 16
_NW = _NC * _NS
_TR = 8


def _sc_body(x_hbm, pe_hbm, o_hbm, peb, xb, lsem, ssem):
    Bn, L, D = x_hbm.shape
    rows = L // _NW
    nt = rows // _TR
    wid = lax.axis_index("s") * _NC + lax.axis_index("c")
    base = wid * rows

    def x_load(k, slot):
        t = k // Bn
        b = k % Bn
        pltpu.async_copy(
            x_hbm.at[b, pl.ds(base + t * _TR, _TR), :], xb.at[slot],
            lsem.at[slot])

    x_load(0, 0)

    def t_body(t, _):
        pltpu.sync_copy(pe_hbm.at[pl.ds(base + t * _TR, _TR), :], peb)
        for b in range(Bn):
            s = b & 1
            k = t * Bn + b
            pltpu.make_async_copy(
                x_hbm.at[0, pl.ds(0, _TR), :], xb.at[s], lsem.at[s]).wait()
            if b == Bn - 1:
                @pl.when(t + 1 < nt)
                def _():
                    pltpu.make_async_copy(
                        xb.at[1 - s], o_hbm.at[0, pl.ds(0, _TR), :],
                        ssem.at[1 - s]).wait()
                    x_load(k + 1, 1 - s)
            elif b == 0:
                @pl.when(t >= 1)
                def _():
                    pltpu.make_async_copy(
                        xb.at[1 - s], o_hbm.at[0, pl.ds(0, _TR), :],
                        ssem.at[1 - s]).wait()
                x_load(k + 1, 1 - s)
            else:
                pltpu.make_async_copy(
                    xb.at[1 - s], o_hbm.at[0, pl.ds(0, _TR), :],
                    ssem.at[1 - s]).wait()
                x_load(k + 1, 1 - s)

            xbs = xb.at[s]
            for r in range(_TR):
                @plsc.parallel_loop(0, D, step=16, unroll=8)
                def _(c):
                    sl = pl.ds(c, 16)
                    xbs[r, sl] = xbs[r, sl] + peb[r, sl]

            pltpu.async_copy(
                xbs, o_hbm.at[b, pl.ds(base + t * _TR, _TR), :], ssem.at[s])
        return 0

    lax.fori_loop(0, nt, t_body, 0)

    pltpu.make_async_copy(
        xb.at[0], o_hbm.at[0, pl.ds(0, _TR), :], ssem.at[0]).wait()
    pltpu.make_async_copy(
        xb.at[1], o_hbm.at[0, pl.ds(0, _TR), :], ssem.at[1]).wait()


def kernel(x, pe):
    B, L, D = x.shape
    return pl.kernel(
        _sc_body,
        out_type=jax.ShapeDtypeStruct((B, L, D), x.dtype),
        mesh=plsc.VectorSubcoreMesh(core_axis_name="c", subcore_axis_name="s"),
        scratch_types=[
            pltpu.VMEM((_TR, D), jnp.float32),
            pltpu.VMEM((2, _TR, D), jnp.float32),
            pltpu.SemaphoreType.DMA((2,)),
            pltpu.SemaphoreType.DMA((2,)),
        ],
        compiler_params=pltpu.CompilerParams(use_tc_tiling_on_sc=True),
    )(x, pe)


# speedup vs baseline: 1.1532x; 1.1532x over previous
import jax
import jax.numpy as jnp
from jax import lax
from jax.experimental import pallas as pl
from jax.experimental.pallas import tpu as pltpu
from jax.experimental.pallas import tpu_sc as plsc

_NC = 2
_NS = 16
_NW = _NC * _NS
_TR = 8
_CW = 1024


def _sc_body(x_hbm, pe_hbm, o_hbm, peb, xb, psem, lsem, ssem):
    Bn, L, D = x_hbm.shape
    rows = L // _NW
    ntr = rows // _TR
    ncq = D // _CW
    wid = lax.axis_index("s") * _NC + lax.axis_index("c")
    rbase = wid * rows

    def issue_loads(r0, c0, slot):
        pltpu.async_copy(
            pe_hbm.at[pl.ds(r0, _TR), pl.ds(c0, _CW)], peb.at[slot],
            psem.at[slot])
        for b in range(Bn):
            pltpu.async_copy(
                x_hbm.at[b, pl.ds(r0, _TR), pl.ds(c0, _CW)],
                xb.at[slot, b], lsem.at[slot])

    def wait_pe(slot):
        pltpu.make_async_copy(
            pe_hbm.at[pl.ds(0, _TR), pl.ds(0, _CW)], peb.at[slot],
            psem.at[slot]).wait()

    def wait_x(slot):
        pltpu.make_async_copy(
            x_hbm.at[pl.ds(0, Bn), pl.ds(0, _TR), pl.ds(0, _CW)],
            xb.at[slot], lsem.at[slot]).wait()

    def drain_stores(slot):
        pltpu.make_async_copy(
            xb.at[slot],
            o_hbm.at[pl.ds(0, Bn), pl.ds(0, _TR), pl.ds(0, _CW)],
            ssem.at[slot]).wait()

    issue_loads(rbase, 0, 0)

    def tr_body(tr, _):
        r0 = rbase + tr * _TR
        for cq in range(ncq):
            s = cq & 1
            c0 = cq * _CW
            wait_pe(s)
            wait_x(s)
            if cq < ncq - 1:
                if cq == 0:
                    @pl.when(tr >= 1)
                    def _():
                        drain_stores(1 - s)
                else:
                    drain_stores(1 - s)
                issue_loads(r0, c0 + _CW, 1 - s)
            else:
                @pl.when(tr + 1 < ntr)
                def _():
                    drain_stores(1 - s)
                    issue_loads(r0 + _TR, 0, 1 - s)

            pes = peb.at[s]
            for r in range(_TR):
                @plsc.parallel_loop(0, _CW, step=16, unroll=4)
                def _(c):
                    sl = pl.ds(c, 16)
                    pv = pes[r, sl]
                    for b in range(Bn):
                        xb[s, b, r, sl] = xb[s, b, r, sl] + pv

            for b in range(Bn):
                pltpu.async_copy(
                    xb.at[s, b],
                    o_hbm.at[b, pl.ds(r0, _TR), pl.ds(c0, _CW)],
                    ssem.at[s])
        return 0

    lax.fori_loop(0, ntr, tr_body, 0)

    drain_stores(0)
    drain_stores(1)


def kernel(x, pe):
    B, L, D = x.shape
    return pl.kernel(
        _sc_body,
        out_type=jax.ShapeDtypeStruct((B, L, D), x.dtype),
        mesh=plsc.VectorSubcoreMesh(core_axis_name="c", subcore_axis_name="s"),
        scratch_types=[
            pltpu.VMEM((2, _TR, _CW), jnp.float32),
            pltpu.VMEM((2, B, _TR, _CW), jnp.float32),
            pltpu.SemaphoreType.DMA((2,)),
            pltpu.SemaphoreType.DMA((2,)),
            pltpu.SemaphoreType.DMA((2,)),
        ],
        compiler_params=pltpu.CompilerParams(use_tc_tiling_on_sc=True),
    )(x, pe)


# speedup vs baseline: 1.1545x; 1.0012x over previous
import jax
import jax.numpy as jnp
from jax import lax
from jax.experimental import pallas as pl
from jax.experimental.pallas import tpu as pltpu
from jax.experimental.pallas import tpu_sc as plsc

_NC = 2
_NS = 16
_NW = _NC * _NS
_TR = 8
_CW = 1024


def _sc_body(x_hbm, pe_hbm, o_hbm, peb, xb, psem, lsem, ssem):
    Bn, L, D = x_hbm.shape
    rows = L // _NW
    ntr = rows // _TR
    ncq = D // _CW
    wid = lax.axis_index("s") * _NC + lax.axis_index("c")
    rbase = wid * rows

    def issue_loads(r0, c0, slot):
        pltpu.async_copy(
            pe_hbm.at[pl.ds(r0, _TR), pl.ds(c0, _CW)], peb.at[slot],
            psem.at[slot])
        pltpu.async_copy(
            x_hbm.at[pl.ds(0, Bn), pl.ds(r0, _TR), pl.ds(c0, _CW)],
            xb.at[slot], lsem.at[slot])

    def wait_pe(slot):
        pltpu.make_async_copy(
            pe_hbm.at[pl.ds(0, _TR), pl.ds(0, _CW)], peb.at[slot],
            psem.at[slot]).wait()

    def wait_x(slot):
        pltpu.make_async_copy(
            x_hbm.at[pl.ds(0, Bn), pl.ds(0, _TR), pl.ds(0, _CW)],
            xb.at[slot], lsem.at[slot]).wait()

    def drain_stores(slot):
        pltpu.make_async_copy(
            xb.at[slot],
            o_hbm.at[pl.ds(0, Bn), pl.ds(0, _TR), pl.ds(0, _CW)],
            ssem.at[slot]).wait()

    issue_loads(rbase, 0, 0)

    def tr_body(tr, _):
        r0 = rbase + tr * _TR
        for cq in range(ncq):
            s = cq & 1
            c0 = cq * _CW
            wait_pe(s)
            wait_x(s)
            if cq < ncq - 1:
                if cq == 0:
                    @pl.when(tr >= 1)
                    def _():
                        drain_stores(1 - s)
                else:
                    drain_stores(1 - s)
                issue_loads(r0, c0 + _CW, 1 - s)
            else:
                @pl.when(tr + 1 < ntr)
                def _():
                    drain_stores(1 - s)
                    issue_loads(r0 + _TR, 0, 1 - s)

            pes = peb.at[s]
            for r in range(_TR):
                @plsc.parallel_loop(0, _CW, step=16, unroll=4)
                def _(c):
                    sl = pl.ds(c, 16)
                    pv = pes[r, sl]
                    for b in range(Bn):
                        xb[s, b, r, sl] = xb[s, b, r, sl] + pv

            pltpu.async_copy(
                xb.at[s],
                o_hbm.at[pl.ds(0, Bn), pl.ds(r0, _TR), pl.ds(c0, _CW)],
                ssem.at[s])
        return 0

    lax.fori_loop(0, ntr, tr_body, 0)

    drain_stores(0)
    drain_stores(1)


def kernel(x, pe):
    B, L, D = x.shape
    return pl.kernel(
        _sc_body,
        out_type=jax.ShapeDtypeStruct((B, L, D), x.dtype),
        mesh=plsc.VectorSubcoreMesh(core_axis_name="c", subcore_axis_name="s"),
        scratch_types=[
            pltpu.VMEM((2, _TR, _CW), jnp.float32),
            pltpu.VMEM((2, B, _TR, _CW), jnp.float32),
            pltpu.SemaphoreType.DMA((2,)),
            pltpu.SemaphoreType.DMA((2,)),
            pltpu.SemaphoreType.DMA((2,)),
        ],
        compiler_params=pltpu.CompilerParams(use_tc_tiling_on_sc=True),
    )(x, pe)
